# Initial kernel scaffold; baseline (speedup 1.0000x reference)
import jax
import jax.numpy as jnp
from jax import lax
from jax.experimental import pallas as pl
from jax.experimental.pallas import tpu as pltpu
from jax.experimental.pallas import tpu_sc as plsc

N = 10000
D = 128
C = 128
DE = 16
NEG = 0.2
NPAD = 10240
E = 320000
NTILES = 32
K = 128
CHUNKS = 79
EPAD = NTILES * CHUNKS * K
TROWS = NPAD // 16


def _prep_body(x_ref, wlt_ref, bl_ref, wrt_ref, br_ref, xl_ref, xr_ref):
    xb = x_ref[...]
    xl_ref[...] = jnp.dot(xb, wlt_ref[...],
                          preferred_element_type=jnp.float32) + bl_ref[...]
    xr_ref[...] = jnp.dot(xb, wrt_ref[...],
                          preferred_element_type=jnp.float32) + br_ref[...]


def _tc_prep(x_pad, WlT, bl, WrT, br):
    blk = 1024
    return pl.pallas_call(
        _prep_body,
        grid=(NPAD // blk,),
        in_specs=[
            pl.BlockSpec((blk, 128), lambda i: (i, 0)),
            pl.BlockSpec((128, 128), lambda i: (0, 0)),
            pl.BlockSpec((1, 128), lambda i: (0, 0)),
            pl.BlockSpec((128, 128), lambda i: (0, 0)),
            pl.BlockSpec((1, 128), lambda i: (0, 0)),
        ],
        out_specs=[
            pl.BlockSpec((blk, 128), lambda i: (i, 0)),
            pl.BlockSpec((blk, 128), lambda i: (i, 0)),
        ],
        out_shape=[jax.ShapeDtypeStruct((NPAD, 128), jnp.float32),
                   jax.ShapeDtypeStruct((NPAD, 128), jnp.float32)],
    )(x_pad, WlT, bl.reshape(1, 128), WrT, br.reshape(1, 128))


def _ea_body(e_ref, wet_ref, o_ref):
    o_ref[...] = jnp.dot(e_ref[...], wet_ref[...],
                         preferred_element_type=jnp.float32)


def _tc_ea(edge_attr_pad, WeT):
    blk = 2048
    return pl.pallas_call(
        _ea_body,
        grid=(EPAD // blk,),
        in_specs=[
            pl.BlockSpec((blk, DE), lambda i: (i, 0)),
            pl.BlockSpec((DE, 128), lambda i: (0, 0)),
        ],
        out_specs=pl.BlockSpec((blk, 128), lambda i: (i, 0)),
        out_shape=jax.ShapeDtypeStruct((EPAD, 128), jnp.float32),
    )(edge_attr_pad, WeT)


def _fin_body_relu(acc_ref, den_ref, bias_ref, o_ref):
    a = acc_ref[0] + acc_ref[1]
    dn = den_ref[0, :, :1] + den_ref[1, :, :1]
    o_ref[...] = jnp.maximum(a / (dn + 1e-16) + bias_ref[...], 0.0)


def _fin_body(acc_ref, den_ref, bias_ref, o_ref):
    a = acc_ref[0] + acc_ref[1]
    dn = den_ref[0, :, :1] + den_ref[1, :, :1]
    o_ref[...] = a / (dn + 1e-16) + bias_ref[...]


def _tc_fin(acc, den, bias, relu):
    blk = 1000
    return pl.pallas_call(
        _fin_body_relu if relu else _fin_body,
        grid=(N // blk,),
        in_specs=[
            pl.BlockSpec((2, blk, 128), lambda i: (0, i, 0)),
            pl.BlockSpec((2, blk, 16), lambda i: (0, i, 0)),
            pl.BlockSpec((1, 128), lambda i: (0, 0)),
        ],
        out_specs=pl.BlockSpec((blk, 128), lambda i: (i, 0)),
        out_shape=jax.ShapeDtypeStruct((N, 128), jnp.float32),
    )(acc, den, bias.reshape(1, 128))


def _sc_edge_body(xl_hbm, xr_hbm, ea_hbm, src_hbm, dst_hbm, att_hbm,
                  zacc_hbm, zden_hbm, accout_hbm, denout_hbm,
                  acc_sp, den_sp, xj, xi, eab, w2d, srcs, dsts, attb):
    cid = lax.axis_index("c")
    sid = lax.axis_index("s")
    wid = sid * 2 + cid
    r0 = sid * TROWS
    pltpu.sync_copy(zacc_hbm.at[pl.ds(r0, TROWS)], acc_sp.at[pl.ds(r0, TROWS)])
    pltpu.sync_copy(zden_hbm.at[pl.ds(r0, TROWS)], den_sp.at[pl.ds(r0, TROWS)])
    pltpu.sync_copy(att_hbm, attb)
    plsc.subcore_barrier()

    att_vecs = [attb[pl.ds(16 * j, 16)] for j in range(8)]

    @pl.loop(0, CHUNKS)
    def _chunk(cidx):
        base = (wid * CHUNKS + cidx) * K
        pltpu.sync_copy(src_hbm.at[pl.ds(base, K)], srcs)
        pltpu.sync_copy(dst_hbm.at[pl.ds(base, K)], dsts)
        pltpu.sync_copy(xl_hbm.at[srcs], xj)
        pltpu.sync_copy(xr_hbm.at[dsts], xi)
        pltpu.sync_copy(ea_hbm.at[pl.ds(base, K)], eab)

        @pl.loop(0, K)
        def _edge(e):
            acc = jnp.zeros((16,), jnp.float32)
            for j in range(8):
                sl = pl.ds(16 * j, 16)
                t = xj[e, sl] + xi[e, sl] + eab[e, sl]
                t = jnp.maximum(t, t * NEG)
                acc = acc + t * att_vecs[j]
            s = jnp.sum(acc)
            wv = jnp.exp(jnp.full((16,), s, jnp.float32))
            w2d[e, :] = wv
            for j in range(8):
                sl = pl.ds(16 * j, 16)
                xj[e, sl] = xj[e, sl] * wv

        pltpu.sync_copy(xj, acc_sp.at[dsts], add=True)
        pltpu.sync_copy(w2d, den_sp.at[dsts], add=True)

    plsc.subcore_barrier()
    pltpu.sync_copy(acc_sp.at[pl.ds(r0, TROWS)],
                    accout_hbm.at[cid, pl.ds(r0, TROWS)])
    pltpu.sync_copy(den_sp.at[pl.ds(r0, TROWS)],
                    denout_hbm.at[cid, pl.ds(r0, TROWS)])


def _sc_edge(xl, xr, ea, src_p, dst_p, attv, zacc, zden):
    mesh = plsc.VectorSubcoreMesh(core_axis_name="c", subcore_axis_name="s")
    kfn = pl.kernel(
        _sc_edge_body,
        out_type=[jax.ShapeDtypeStruct((2, NPAD, 128), jnp.float32),
                  jax.ShapeDtypeStruct((2, NPAD, 16), jnp.float32)],
        mesh=mesh,
        scratch_types=[
            pltpu.VMEM_SHARED((NPAD, 128), jnp.float32),
            pltpu.VMEM_SHARED((NPAD, 16), jnp.float32),
            pltpu.VMEM((K, 128), jnp.float32),
            pltpu.VMEM((K, 128), jnp.float32),
            pltpu.VMEM((K, 128), jnp.float32),
            pltpu.VMEM((K, 16), jnp.float32),
            pltpu.VMEM((K,), jnp.int32),
            pltpu.VMEM((K,), jnp.int32),
            pltpu.VMEM((128,), jnp.float32),
        ],
    )
    return kfn(xl, xr, ea, src_p, dst_p, attv, zacc, zden)


def kernel(x, edge_index, edge_attr,
           Wl0, bl0, Wr0, br0, We0, att0, bias0,
           Wl1, bl1, Wr1, br1, We1, att1, bias1):
    src = edge_index[0].astype(jnp.int32)
    dst = edge_index[1].astype(jnp.int32)
    pe = EPAD - E
    src_p = jnp.concatenate([src, jnp.zeros((pe,), jnp.int32)])
    dst_p = jnp.concatenate([dst, jnp.full((pe,), N, jnp.int32)])
    ea_p = jnp.concatenate(
        [edge_attr, jnp.zeros((pe, DE), jnp.float32)], axis=0)
    x_pad = jnp.concatenate(
        [x, jnp.zeros((NPAD - N, D), jnp.float32)], axis=0)
    zacc = jnp.zeros((NPAD, 128), jnp.float32)
    zden = jnp.zeros((NPAD, 16), jnp.float32)

    ea0 = _tc_ea(ea_p, We0.T)
    ea1 = _tc_ea(ea_p, We1.T)

    xl0, xr0 = _tc_prep(x_pad, Wl0.T, bl0, Wr0.T, br0)
    acc0, den0 = _sc_edge(xl0, xr0, ea0, src_p, dst_p,
                          att0.reshape(128), zacc, zden)
    h = _tc_fin(acc0, den0, bias0, relu=True)

    h_pad = jnp.concatenate(
        [h, jnp.zeros((NPAD - N, 128), jnp.float32)], axis=0)
    xl1, xr1 = _tc_prep(h_pad, Wl1.T, bl1, Wr1.T, br1)
    acc1, den1 = _sc_edge(xl1, xr1, ea1, src_p, dst_p,
                          att1.reshape(128), zacc, zden)
    return _tc_fin(acc1, den1, bias1, relu=False)


# speedup vs baseline: 4.0476x; 4.0476x over previous
import dataclasses

import jax
import jax.numpy as jnp
from jax import lax
from jax.experimental import pallas as pl
from jax.experimental.pallas import tpu as pltpu
from jax.experimental.pallas import tpu_sc as plsc

N = 10000
D = 128
C = 128
DE = 16
NEG = 0.2
NPAD = 10240
E = 320000
NTILES = 32
K = 128
CHUNKS = 79
EPAD = NTILES * CHUNKS * K
TROWS = NPAD // 16


def _prep_body(x_ref, wlt_ref, bl_ref, wrt_ref, br_ref, xl_ref, xr_ref):
    xb = x_ref[...]
    xl_ref[...] = jnp.dot(xb, wlt_ref[...],
                          preferred_element_type=jnp.float32) + bl_ref[...]
    xr_ref[...] = jnp.dot(xb, wrt_ref[...],
                          preferred_element_type=jnp.float32) + br_ref[...]


def _tc_prep(x_pad, WlT, bl, WrT, br):
    blk = 1024
    return pl.pallas_call(
        _prep_body,
        grid=(NPAD // blk,),
        in_specs=[
            pl.BlockSpec((blk, 128), lambda i: (i, 0)),
            pl.BlockSpec((128, 128), lambda i: (0, 0)),
            pl.BlockSpec((1, 128), lambda i: (0, 0)),
            pl.BlockSpec((128, 128), lambda i: (0, 0)),
            pl.BlockSpec((1, 128), lambda i: (0, 0)),
        ],
        out_specs=[
            pl.BlockSpec((blk, 128), lambda i: (i, 0)),
            pl.BlockSpec((blk, 128), lambda i: (i, 0)),
        ],
        out_shape=[jax.ShapeDtypeStruct((NPAD, 128), jnp.float32),
                   jax.ShapeDtypeStruct((NPAD, 128), jnp.float32)],
    )(x_pad, WlT, bl.reshape(1, 128), WrT, br.reshape(1, 128))


def _ea_body(e_ref, wet_ref, o_ref):
    o_ref[...] = jnp.dot(e_ref[...], wet_ref[...],
                         preferred_element_type=jnp.float32)


def _tc_ea(edge_attr_pad, WeT):
    blk = 2048
    return pl.pallas_call(
        _ea_body,
        grid=(EPAD // blk,),
        in_specs=[
            pl.BlockSpec((blk, DE), lambda i: (i, 0)),
            pl.BlockSpec((DE, 128), lambda i: (0, 0)),
        ],
        out_specs=pl.BlockSpec((blk, 128), lambda i: (i, 0)),
        out_shape=jax.ShapeDtypeStruct((EPAD, 128), jnp.float32),
    )(edge_attr_pad, WeT)


def _fin_body_relu(acc_ref, den_ref, bias_ref, o_ref):
    a = acc_ref[0] + acc_ref[1]
    dn = den_ref[0, :, :1] + den_ref[1, :, :1]
    o_ref[...] = jnp.maximum(a / (dn + 1e-16) + bias_ref[...], 0.0)


def _fin_body(acc_ref, den_ref, bias_ref, o_ref):
    a = acc_ref[0] + acc_ref[1]
    dn = den_ref[0, :, :1] + den_ref[1, :, :1]
    o_ref[...] = a / (dn + 1e-16) + bias_ref[...]


def _tc_fin(acc, den, bias, relu):
    blk = 1000
    return pl.pallas_call(
        _fin_body_relu if relu else _fin_body,
        grid=(N // blk,),
        in_specs=[
            pl.BlockSpec((2, blk, 128), lambda i: (0, i, 0)),
            pl.BlockSpec((2, blk, 16), lambda i: (0, i, 0)),
            pl.BlockSpec((1, 128), lambda i: (0, 0)),
        ],
        out_specs=pl.BlockSpec((blk, 128), lambda i: (i, 0)),
        out_shape=jax.ShapeDtypeStruct((N, 128), jnp.float32),
    )(acc, den, bias.reshape(1, 128))


def _sc_alpha_body(xl_hbm, xr_hbm, ea_hbm, src_hbm, dst_hbm, att_hbm,
                   zden_hbm, w_hbm, denout_hbm,
                   den_sp, xj, xi, eab, w2d, srcs, dsts, attb):
    cid = lax.axis_index("c")
    sid = lax.axis_index("s")
    wid = sid * 2 + cid
    r0 = sid * TROWS
    pltpu.sync_copy(zden_hbm.at[pl.ds(r0, TROWS)], den_sp.at[pl.ds(r0, TROWS)])
    pltpu.sync_copy(att_hbm, attb)
    plsc.subcore_barrier()

    att_vecs = [attb[pl.ds(16 * j, 16)] for j in range(8)]

    @pl.loop(0, CHUNKS)
    def _chunk(cidx):
        base = (wid * CHUNKS + cidx) * K
        pltpu.sync_copy(src_hbm.at[pl.ds(base, K)], srcs)
        pltpu.sync_copy(dst_hbm.at[pl.ds(base, K)], dsts)
        pltpu.sync_copy(xl_hbm.at[srcs], xj)
        pltpu.sync_copy(xr_hbm.at[dsts], xi)
        pltpu.sync_copy(ea_hbm.at[pl.ds(base, K)], eab)

        @pl.loop(0, K)
        def _edge(e):
            acc = jnp.zeros((16,), jnp.float32)
            for j in range(8):
                sl = pl.ds(16 * j, 16)
                t = xj[e, sl] + xi[e, sl] + eab[e, sl]
                t = jnp.maximum(t, t * NEG)
                acc = acc + t * att_vecs[j]
            s = jnp.sum(acc)
            w2d[e, :] = jnp.exp(jnp.full((16,), s, jnp.float32))

        pltpu.sync_copy(w2d, den_sp.at[dsts], add=True)
        pltpu.sync_copy(w2d, w_hbm.at[pl.ds(base, K)])

    plsc.subcore_barrier()
    pltpu.sync_copy(den_sp.at[pl.ds(r0, TROWS)],
                    denout_hbm.at[cid, pl.ds(r0, TROWS)])


def _sc_agg_body(xl_hbm, src_hbm, dst_hbm, w_hbm, zacc_hbm, accout_hbm,
                 acc_sp, xj, w2d, srcs, dsts):
    cid = lax.axis_index("c")
    sid = lax.axis_index("s")
    wid = sid * 2 + cid
    r0 = sid * TROWS
    pltpu.sync_copy(zacc_hbm.at[pl.ds(r0, TROWS)], acc_sp.at[pl.ds(r0, TROWS)])
    plsc.subcore_barrier()

    @pl.loop(0, CHUNKS)
    def _chunk(cidx):
        base = (wid * CHUNKS + cidx) * K
        pltpu.sync_copy(src_hbm.at[pl.ds(base, K)], srcs)
        pltpu.sync_copy(dst_hbm.at[pl.ds(base, K)], dsts)
        pltpu.sync_copy(xl_hbm.at[srcs], xj)
        pltpu.sync_copy(w_hbm.at[pl.ds(base, K)], w2d)

        @pl.loop(0, K)
        def _edge(e):
            wv = w2d[e, :]
            for j in range(8):
                sl = pl.ds(16 * j, 16)
                xj[e, sl] = xj[e, sl] * wv

        pltpu.sync_copy(xj, acc_sp.at[dsts], add=True)

    plsc.subcore_barrier()
    pltpu.sync_copy(acc_sp.at[pl.ds(r0, TROWS)],
                    accout_hbm.at[cid, pl.ds(r0, TROWS)])


def _sc_compiler_params():
    cp = pltpu.CompilerParams()
    if "needs_layout_passes" in pltpu.CompilerParams.__dataclass_fields__:
        cp = dataclasses.replace(cp, needs_layout_passes=False)
    return cp


def _sc_edge(xl, xr, ea, src_p, dst_p, attv, zacc, zden):
    mesh = plsc.VectorSubcoreMesh(core_axis_name="c", subcore_axis_name="s")
    alpha_fn = pl.kernel(
        _sc_alpha_body,
        out_type=[jax.ShapeDtypeStruct((EPAD, 16), jnp.float32),
                  jax.ShapeDtypeStruct((2, NPAD, 16), jnp.float32)],
        mesh=mesh,
        scratch_types=[
            pltpu.VMEM_SHARED((NPAD, 16), jnp.float32),
            pltpu.VMEM((K, 128), jnp.float32),
            pltpu.VMEM((K, 128), jnp.float32),
            pltpu.VMEM((K, 128), jnp.float32),
            pltpu.VMEM((K, 16), jnp.float32),
            pltpu.VMEM((K,), jnp.int32),
            pltpu.VMEM((K,), jnp.int32),
            pltpu.VMEM((128,), jnp.float32),
        ],
        compiler_params=_sc_compiler_params(),
    )
    w, den = alpha_fn(xl, xr, ea, src_p, dst_p, attv, zden)

    agg_fn = pl.kernel(
        _sc_agg_body,
        out_type=jax.ShapeDtypeStruct((2, NPAD, 128), jnp.float32),
        mesh=mesh,
        scratch_types=[
            pltpu.VMEM_SHARED((NPAD, 128), jnp.float32),
            pltpu.VMEM((K, 128), jnp.float32),
            pltpu.VMEM((K, 16), jnp.float32),
            pltpu.VMEM((K,), jnp.int32),
            pltpu.VMEM((K,), jnp.int32),
        ],
        compiler_params=_sc_compiler_params(),
    )
    acc = agg_fn(xl, src_p, dst_p, w, zacc)
    return acc, den


def kernel(x, edge_index, edge_attr,
           Wl0, bl0, Wr0, br0, We0, att0, bias0,
           Wl1, bl1, Wr1, br1, We1, att1, bias1):
    src = edge_index[0].astype(jnp.int32)
    dst = edge_index[1].astype(jnp.int32)
    pe = EPAD - E
    src_p = jnp.concatenate([src, jnp.zeros((pe,), jnp.int32)])
    dst_p = jnp.concatenate([dst, jnp.full((pe,), N, jnp.int32)])
    ea_p = jnp.concatenate(
        [edge_attr, jnp.zeros((pe, DE), jnp.float32)], axis=0)
    x_pad = jnp.concatenate(
        [x, jnp.zeros((NPAD - N, D), jnp.float32)], axis=0)
    zacc = jnp.zeros((NPAD, 128), jnp.float32)
    zden = jnp.zeros((NPAD, 16), jnp.float32)

    ea0 = _tc_ea(ea_p, We0.T)
    ea1 = _tc_ea(ea_p, We1.T)

    xl0, xr0 = _tc_prep(x_pad, Wl0.T, bl0, Wr0.T, br0)
    acc0, den0 = _sc_edge(xl0, xr0, ea0, src_p, dst_p,
                          att0.reshape(128), zacc, zden)
    h = _tc_fin(acc0, den0, bias0, relu=True)

    h_pad = jnp.concatenate(
        [h, jnp.zeros((NPAD - N, 128), jnp.float32)], axis=0)
    xl1, xr1 = _tc_prep(h_pad, Wl1.T, bl1, Wr1.T, br1)
    acc1, den1 = _sc_edge(xl1, xr1, ea1, src_p, dst_p,
                          att1.reshape(128), zacc, zden)
    return _tc_fin(acc1, den1, bias1, relu=False)
